# Initial kernel scaffold; baseline (speedup 1.0000x reference)
import jax
import jax.numpy as jnp
from jax.experimental import pallas as pl


def kernel(point_cloud):
    raise NotImplementedError("write your pallas kernel here")


# speedup vs baseline: 15.9548x; 15.9548x over previous
import jax
import jax.numpy as jnp
from jax.experimental import pallas as pl

_K = 10
_N = 2048
_RB = 256
_BIG = 3.0e38


def _knn_avg_kernel(pct_ref, rows_ref, out_ref):
    q3 = pct_ref[0]
    p = rows_ref[0]
    d2 = jnp.zeros((_RB, _N), jnp.float32)
    for c in range(3):
        pc_col = p[:, c:(c + 1)]
        qc_row = q3[c:(c + 1), :]
        diff = pc_col - qc_row
        d2 = d2 + diff * diff
    d2 = jnp.maximum(d2, 1e-12)
    acc = jnp.zeros((_RB, 1), jnp.float32)
    taken = jnp.zeros((_RB, 1), jnp.float32)
    first = None
    kk = float(_K + 1)
    for t in range(_K + 1):
        m = jnp.min(d2, axis=1, keepdims=True)
        eq = d2 == m
        cnt = jnp.sum(eq.astype(jnp.float32), axis=1, keepdims=True)
        take = jnp.minimum(cnt, kk - taken)
        root = jnp.sqrt(m)
        acc = acc + take * root
        taken = taken + take
        if t == 0:
            first = root
        if t < _K:
            d2 = jnp.where(eq, _BIG, d2)
    avg = (acc - first) * jnp.float32(1.0 / _K)
    out_ref[0, :, :] = avg


def kernel(point_cloud):
    B, N, D = point_cloud.shape
    pct = jnp.transpose(point_cloud, (0, 2, 1))
    nrb = N // _RB
    out = pl.pallas_call(
        _knn_avg_kernel,
        grid=(B, nrb),
        in_specs=[
            pl.BlockSpec((1, D, N), lambda b, r: (b, 0, 0)),
            pl.BlockSpec((1, _RB, D), lambda b, r: (b, r, 0)),
        ],
        out_specs=pl.BlockSpec((1, _RB, 1), lambda b, r: (b * nrb + r, 0, 0)),
        out_shape=jax.ShapeDtypeStruct((B * nrb, _RB, 1), jnp.float32),
    )(pct, point_cloud)
    avg = out.reshape(B, N)
    var = jnp.var(avg, axis=-1, ddof=1)
    return jnp.mean(var)


# speedup vs baseline: 19.4666x; 1.2201x over previous
import jax
import jax.numpy as jnp
from jax.experimental import pallas as pl

_K = 10
_N = 2048
_RB = 256
_IDX_BITS = 11
_IDX_MASK = (1 << _IDX_BITS) - 1
_IBIG = 0x7F000000


def _knn_avg_kernel(pct_ref, rows_ref, out_ref):
    r = pl.program_id(1)
    q3 = pct_ref[0]
    p = rows_ref[0]
    sq_p = jnp.sum(p * p, axis=1, keepdims=True)
    sq_q = jnp.sum(q3 * q3, axis=0, keepdims=True)
    dot = jax.lax.dot_general(p, q3, (((1,), (0,)), ((), ())),
                              preferred_element_type=jnp.float32)
    d2 = jnp.maximum(sq_p + sq_q - 2.0 * dot, 1e-12)
    keys = jax.lax.bitcast_convert_type(d2, jnp.int32)
    col = jax.lax.broadcasted_iota(jnp.int32, (_RB, _N), 1)
    keys = (keys & ~_IDX_MASK) | col
    row = jax.lax.broadcasted_iota(jnp.int32, (_RB, _N), 0) + r * _RB
    keys = jnp.where(col == row, _IBIG, keys)
    acc = jnp.zeros((_RB, 1), jnp.float32)
    for t in range(_K):
        m = jnp.min(keys, axis=1, keepdims=True)
        val = jax.lax.bitcast_convert_type(m & ~_IDX_MASK, jnp.float32)
        acc = acc + jnp.sqrt(val)
        if t < _K - 1:
            keys = jnp.where(keys == m, _IBIG, keys)
    out_ref[0, :, :] = acc * (1.0 / _K)


def kernel(point_cloud):
    B, N, D = point_cloud.shape
    pct = jnp.transpose(point_cloud, (0, 2, 1))
    nrb = N // _RB
    out = pl.pallas_call(
        _knn_avg_kernel,
        grid=(B, nrb),
        in_specs=[
            pl.BlockSpec((1, D, N), lambda b, r: (b, 0, 0)),
            pl.BlockSpec((1, _RB, D), lambda b, r: (b, r, 0)),
        ],
        out_specs=pl.BlockSpec((1, _RB, 1), lambda b, r: (b * nrb + r, 0, 0)),
        out_shape=jax.ShapeDtypeStruct((B * nrb, _RB, 1), jnp.float32),
    )(pct, point_cloud)
    avg = out.reshape(B, N)
    var = jnp.var(avg, axis=-1, ddof=1)
    return jnp.mean(var)


# speedup vs baseline: 28.3474x; 1.4562x over previous
import jax
import jax.numpy as jnp
from jax.experimental import pallas as pl

_K = 10
_N = 2048
_RB = 256
_IDX_BITS = 11
_IDX_MASK = (1 << _IDX_BITS) - 1
_IBIG = 0x7F000000


def _knn_avg_kernel(pct_ref, rows_ref, out_ref):
    r = pl.program_id(1)
    q3 = pct_ref[0]
    p = rows_ref[0]
    sq_p = jnp.sum(p * p, axis=1, keepdims=True)
    sq_q = jnp.sum(q3 * q3, axis=0, keepdims=True)
    dot = jax.lax.dot_general(p, q3, (((1,), (0,)), ((), ())),
                              preferred_element_type=jnp.float32)
    d2 = jnp.maximum(sq_p + sq_q - 2.0 * dot, 1e-12)
    ikeys = jax.lax.bitcast_convert_type(d2, jnp.int32)
    col = jax.lax.broadcasted_iota(jnp.int32, (_RB, _N), 1)
    ikeys = (ikeys & ~_IDX_MASK) | col
    row = jax.lax.broadcasted_iota(jnp.int32, (_RB, _N), 0) + r * _RB
    ikeys = jnp.where(col == row, _IBIG, ikeys)
    keys = jax.lax.bitcast_convert_type(ikeys, jnp.float32)
    fbig = jax.lax.bitcast_convert_type(
        jnp.full((1, 1), _IBIG, jnp.int32), jnp.float32)
    acc = jnp.zeros((_RB, 1), jnp.float32)
    for t in range(_K):
        m = jnp.min(keys, axis=1, keepdims=True)
        mi = jax.lax.bitcast_convert_type(m, jnp.int32)
        val = jax.lax.bitcast_convert_type(mi & ~_IDX_MASK, jnp.float32)
        acc = acc + jnp.sqrt(val)
        if t < _K - 1:
            keys = jnp.where(keys == m, fbig, keys)
    out_ref[0, :, :] = acc * (1.0 / _K)


def kernel(point_cloud):
    B, N, D = point_cloud.shape
    pct = jnp.transpose(point_cloud, (0, 2, 1))
    nrb = N // _RB
    out = pl.pallas_call(
        _knn_avg_kernel,
        grid=(B, nrb),
        in_specs=[
            pl.BlockSpec((1, D, N), lambda b, r: (b, 0, 0)),
            pl.BlockSpec((1, _RB, D), lambda b, r: (b, r, 0)),
        ],
        out_specs=pl.BlockSpec((1, _RB, 1), lambda b, r: (b * nrb + r, 0, 0)),
        out_shape=jax.ShapeDtypeStruct((B * nrb, _RB, 1), jnp.float32),
    )(pct, point_cloud)
    avg = out.reshape(B, N)
    var = jnp.var(avg, axis=-1, ddof=1)
    return jnp.mean(var)
